# baseline (speedup 1.0000x reference)
import math

import jax
import jax.numpy as jnp
from jax.experimental import pallas as pl
from jax.experimental.pallas import tpu as pltpu

NUM_FEATURES = 128
NUM_PIDS = 100000
NUM_CQ = 5000
OIM_SCALAR = 30.0
BATCH = 128
NUM_STEPS = 5
HALF = NUM_PIDS // 2
BLK = HALF // NUM_STEPS
IGNORE_INDEX = 5554
LOG2E = math.log2(math.e)
LN2 = math.log(2.0)


def _oim_kernel(x_ref, lab_ref, luta_ref, cq_ref, relca_ref, relcb_ref,
                rel_cq_ref, lutb_hbm, out_ref, s_ref, lsc_ref, buf, sems):
    i = pl.program_id(0)
    x = x_ref[...]
    labels = lab_ref[...]

    def b_copy(k):
        return pltpu.make_async_copy(
            lutb_hbm.at[pl.ds(k * BLK, BLK), :], buf.at[k % 2], sems.at[k % 2])

    def scores2(w, relc):
        lg = jax.lax.dot_general(
            x, w, (((1,), (1,)), ((), ())),
            preferred_element_type=jnp.float32,
            precision=jax.lax.Precision.DEFAULT)
        return lg * relc

    def accum(w, relc, base):
        y = scores2(w, relc)
        se = jnp.sum(jnp.exp2(y), axis=1, keepdims=True)
        col = jax.lax.broadcasted_iota(jnp.int32, y.shape, 1)
        hit = col == labels - base
        lsum = jnp.sum(jnp.where(hit, y, 0.0), axis=1, keepdims=True)
        return se, lsum

    @pl.when(i == 0)
    def _init():
        pltpu.make_async_copy(
            lutb_hbm.at[pl.ds(HALF, BLK), :], buf.at[0], sems.at[0]).start()
        pltpu.make_async_copy(
            lutb_hbm.at[pl.ds(HALF + BLK, BLK), :], buf.at[1],
            sems.at[1]).start()
        yc = scores2(cq_ref[...], rel_cq_ref[...])
        s_ref[...] = jnp.sum(jnp.exp2(yc), axis=1, keepdims=True)
        lsc_ref[...] = jnp.zeros_like(lsc_ref)

    se_a, lsum_a = accum(luta_ref[...], relca_ref[0], i * BLK)

    slot = jax.lax.rem(i, 2)
    pltpu.make_async_copy(
        lutb_hbm.at[pl.ds(HALF + i * BLK, BLK), :], buf.at[slot],
        sems.at[slot]).wait()
    se_b, lsum_b = accum(buf[slot], relcb_ref[i], HALF + i * BLK)

    s_ref[...] += se_a + se_b
    lsc_ref[...] += lsum_a + lsum_b

    @pl.when(i < NUM_STEPS - 2)
    def _next_copy():
        pltpu.make_async_copy(
            lutb_hbm.at[pl.ds(HALF + (i + 2) * BLK, BLK), :], buf.at[slot],
            sems.at[slot]).start()

    @pl.when(i == NUM_STEPS - 1)
    def _finish():
        lse = jnp.log(s_ref[...])
        nll = lse - lsc_ref[...] * LN2
        valid = (labels != IGNORE_INDEX).astype(jnp.float32)
        loss = (jnp.sum(nll * valid, keepdims=True)
                / jnp.maximum(jnp.sum(valid, keepdims=True), 1.0))
        out_ref[...] = loss.reshape(1, 1)


def kernel(inputs, roi_label, roi_ious, lut, cq, reliability):
    del roi_ious
    labels = (roi_label.reshape(-1) - 1).astype(jnp.int32).reshape(BATCH, 1)
    relc = reliability * jnp.float32(OIM_SCALAR * LOG2E)
    relca = relc[:HALF].reshape(NUM_STEPS, 1, BLK)
    relcb = relc[HALF:NUM_PIDS].reshape(NUM_STEPS, 1, BLK)
    rel_cq = relc[NUM_PIDS:].reshape(1, NUM_CQ)
    vmem = pltpu.MemorySpace.VMEM
    out = pl.pallas_call(
        _oim_kernel,
        grid=(NUM_STEPS,),
        in_specs=[
            pl.BlockSpec((BATCH, NUM_FEATURES), lambda i: (0, 0)),
            pl.BlockSpec((BATCH, 1), lambda i: (0, 0)),
            pl.BlockSpec((BLK, NUM_FEATURES), lambda i: (i, 0)),
            pl.BlockSpec((NUM_CQ, NUM_FEATURES), lambda i: (0, 0)),
            pl.BlockSpec((1, 1, BLK), lambda i: (i, 0, 0)),
            pl.BlockSpec(memory_space=vmem),
            pl.BlockSpec((1, NUM_CQ), lambda i: (0, 0)),
            pl.BlockSpec(memory_space=pltpu.MemorySpace.HBM),
        ],
        out_specs=pl.BlockSpec((1, 1), lambda i: (0, 0)),
        out_shape=jax.ShapeDtypeStruct((1, 1), jnp.float32),
        scratch_shapes=[
            pltpu.VMEM((BATCH, 1), jnp.float32),
            pltpu.VMEM((BATCH, 1), jnp.float32),
            pltpu.VMEM((2, BLK, NUM_FEATURES), jnp.float32),
            pltpu.SemaphoreType.DMA((2,)),
        ],
    )(inputs, labels, lut, cq, relca, relcb, rel_cq, lut)
    return out[0, 0]


# speedup vs baseline: 1.0143x; 1.0143x over previous
import math

import jax
import jax.numpy as jnp
from jax.experimental import pallas as pl
from jax.experimental.pallas import tpu as pltpu

NUM_FEATURES = 128
NUM_PIDS = 100000
NUM_CQ = 5000
OIM_SCALAR = 30.0
BATCH = 128
BLK = 25000
NUM_BLOCKS = NUM_PIDS // BLK
IGNORE_INDEX = 5554
LOG2E = math.log2(math.e)
LN2 = math.log(2.0)


def _oim_kernel(x_ref, lab_ref, lut_ref, cq_ref, relc_lut_ref, relc_cq_ref,
                out_ref, s_ref, lsc_ref):
    i = pl.program_id(0)
    x = x_ref[...]
    labels = lab_ref[...]

    def scores2(w, relc):
        lg = jax.lax.dot_general(
            x, w, (((1,), (1,)), ((), ())),
            preferred_element_type=jnp.float32,
            precision=jax.lax.Precision.DEFAULT)
        return lg * relc

    @pl.when(i == 0)
    def _init():
        ys = scores2(cq_ref[...], relc_cq_ref[...])
        s_ref[...] = jnp.sum(jnp.exp2(ys), axis=1, keepdims=True)
        lsc_ref[...] = jnp.zeros_like(lsc_ref)

    y = scores2(lut_ref[...], relc_lut_ref[0])
    s_ref[...] += jnp.sum(jnp.exp2(y), axis=1, keepdims=True)

    col = jax.lax.broadcasted_iota(jnp.int32, (BATCH, BLK), 1)
    hit = col == labels - i * BLK
    lsc_ref[...] += jnp.sum(jnp.where(hit, y, 0.0), axis=1, keepdims=True)

    @pl.when(i == NUM_BLOCKS - 1)
    def _finish():
        lse = jnp.log(s_ref[...])
        nll = lse - lsc_ref[...] * LN2
        valid = (labels != IGNORE_INDEX).astype(jnp.float32)
        loss = (jnp.sum(nll * valid, keepdims=True)
                / jnp.maximum(jnp.sum(valid, keepdims=True), 1.0))
        out_ref[...] = loss.reshape(1, 1)


def kernel(inputs, roi_label, roi_ious, lut, cq, reliability):
    del roi_ious
    labels = (roi_label.reshape(-1) - 1).astype(jnp.int32).reshape(BATCH, 1)
    relc = reliability * jnp.float32(OIM_SCALAR * LOG2E)
    relc_lut = relc[:NUM_PIDS].reshape(NUM_BLOCKS, 1, BLK)
    relc_cq = relc[NUM_PIDS:].reshape(1, NUM_CQ)

    out = pl.pallas_call(
        _oim_kernel,
        grid=(NUM_BLOCKS,),
        in_specs=[
            pl.BlockSpec((BATCH, NUM_FEATURES), lambda i: (0, 0)),
            pl.BlockSpec((BATCH, 1), lambda i: (0, 0)),
            pl.BlockSpec((BLK, NUM_FEATURES), lambda i: (i, 0)),
            pl.BlockSpec((NUM_CQ, NUM_FEATURES), lambda i: (0, 0)),
            pl.BlockSpec((1, 1, BLK), lambda i: (i, 0, 0)),
            pl.BlockSpec((1, NUM_CQ), lambda i: (0, 0)),
        ],
        out_specs=pl.BlockSpec((1, 1), lambda i: (0, 0)),
        out_shape=jax.ShapeDtypeStruct((1, 1), jnp.float32),
        scratch_shapes=[
            pltpu.VMEM((BATCH, 1), jnp.float32),
            pltpu.VMEM((BATCH, 1), jnp.float32),
        ],
    )(inputs, labels, lut, cq, relc_lut, relc_cq)
    return out[0, 0]
